# initial kernel scaffold (unmeasured)
import jax
import jax.numpy as jnp
from jax import lax
from jax.experimental import pallas as pl
from jax.experimental.pallas import tpu as pltpu

N_Z = 4
N_EXP_LOCAL = 2
D_MODEL = 1024
D_FF = 2048
T_PER = 1024
A_COLS = 128


def kernel(x, assign, W1, W2):
    a_cols = jnp.broadcast_to(
        assign.astype(jnp.float32)[:, None], (T_PER, A_COLS)
    )
    x_aug = jnp.concatenate([x, a_cols], axis=1)

    def body(
        xaug_ref,
        w1_ref,
        w2_ref,
        out_ref,
        gather_ref,
        partials_ref,
        recv_ref,
        ag_send,
        ag_recv,
        rs_send,
        rs_recv,
        copy_sem,
    ):
        my_x = lax.axis_index("x")
        my_y = lax.axis_index("y")
        my_z = lax.axis_index("z")

        barrier_sem = pltpu.get_barrier_semaphore()
        for j in range(N_Z - 1):
            q = (my_z + 1 + j) % N_Z
            pl.semaphore_signal(
                barrier_sem,
                inc=1,
                device_id=(my_x, my_y, q),
                device_id_type=pl.DeviceIdType.MESH,
            )
        pl.semaphore_wait(barrier_sem, N_Z - 1)

        ag_rdmas = []
        for j in range(N_Z - 1):
            q = (my_z + 1 + j) % N_Z
            r = pltpu.make_async_remote_copy(
                src_ref=xaug_ref,
                dst_ref=gather_ref.at[my_z],
                send_sem=ag_send.at[q],
                recv_sem=ag_recv.at[my_z],
                device_id=(my_x, my_y, q),
                device_id_type=pl.DeviceIdType.MESH,
            )
            r.start()
            ag_rdmas.append(r)

        cp = pltpu.make_async_copy(xaug_ref, gather_ref.at[my_z], copy_sem)
        cp.start()
        cp.wait()

        for j in range(N_Z - 1):
            p = (my_z + 1 + j) % N_Z
            w = pltpu.make_async_remote_copy(
                src_ref=xaug_ref,
                dst_ref=gather_ref.at[p],
                send_sem=ag_send.at[p],
                recv_sem=ag_recv.at[p],
                device_id=(my_x, my_y, p),
                device_id_type=pl.DeviceIdType.MESH,
            )
            w.wait_recv()

        for j in range(N_Z):
            c = (my_z + 1 + j) % N_Z
            xa = gather_ref[c]
            feat = xa[:, :D_MODEL]
            acol = xa[:, D_MODEL : D_MODEL + 1]
            acc = jnp.zeros((T_PER, D_MODEL), dtype=jnp.float32)
            for le in range(N_EXP_LOCAL):
                e = (N_EXP_LOCAL * my_z + le).astype(jnp.float32)
                mask = (acol == e).astype(jnp.float32)
                h = jnp.maximum(
                    jnp.dot(
                        feat * mask,
                        w1_ref[le],
                        preferred_element_type=jnp.float32,
                    ),
                    0.0,
                )
                acc = acc + jnp.dot(
                    h, w2_ref[le], preferred_element_type=jnp.float32
                )
            if j < N_Z - 1:
                partials_ref[c] = acc
                r = pltpu.make_async_remote_copy(
                    src_ref=partials_ref.at[c],
                    dst_ref=recv_ref.at[my_z],
                    send_sem=rs_send.at[c],
                    recv_sem=rs_recv.at[my_z],
                    device_id=(my_x, my_y, c),
                    device_id_type=pl.DeviceIdType.MESH,
                )
                r.start()
            else:
                out_ref[...] = acc

        total = out_ref[...]
        for j in range(N_Z - 1):
            p = (my_z + 1 + j) % N_Z
            w = pltpu.make_async_remote_copy(
                src_ref=partials_ref.at[p],
                dst_ref=recv_ref.at[p],
                send_sem=rs_send.at[p],
                recv_sem=rs_recv.at[p],
                device_id=(my_x, my_y, p),
                device_id_type=pl.DeviceIdType.MESH,
            )
            w.wait_recv()
            total = total + recv_ref[p]
        out_ref[...] = total

        for j in range(N_Z - 1):
            q = (my_z + 1 + j) % N_Z
            ws = pltpu.make_async_remote_copy(
                src_ref=xaug_ref,
                dst_ref=gather_ref.at[my_z],
                send_sem=ag_send.at[q],
                recv_sem=ag_recv.at[my_z],
                device_id=(my_x, my_y, q),
                device_id_type=pl.DeviceIdType.MESH,
            )
            ws.wait_send()
            wr = pltpu.make_async_remote_copy(
                src_ref=partials_ref.at[q],
                dst_ref=recv_ref.at[my_z],
                send_sem=rs_send.at[q],
                recv_sem=rs_recv.at[my_z],
                device_id=(my_x, my_y, q),
                device_id_type=pl.DeviceIdType.MESH,
            )
            wr.wait_send()

    out_shape = jax.ShapeDtypeStruct((T_PER, D_MODEL), jnp.float32)
    return pl.pallas_call(
        body,
        out_shape=out_shape,
        in_specs=[
            pl.BlockSpec(memory_space=pltpu.VMEM),
            pl.BlockSpec(memory_space=pltpu.VMEM),
            pl.BlockSpec(memory_space=pltpu.VMEM),
        ],
        out_specs=pl.BlockSpec(memory_space=pltpu.VMEM),
        scratch_shapes=[
            pltpu.VMEM((N_Z, T_PER, D_MODEL + A_COLS), jnp.float32),
            pltpu.VMEM((N_Z, T_PER, D_MODEL), jnp.float32),
            pltpu.VMEM((N_Z, T_PER, D_MODEL), jnp.float32),
            pltpu.SemaphoreType.DMA((N_Z,)),
            pltpu.SemaphoreType.DMA((N_Z,)),
            pltpu.SemaphoreType.DMA((N_Z,)),
            pltpu.SemaphoreType.DMA((N_Z,)),
            pltpu.SemaphoreType.DMA,
        ],
        compiler_params=pltpu.CompilerParams(collective_id=0),
    )(x_aug, W1, W2)


# baseline (device time: 228745 ns/iter reference)
import jax
import jax.numpy as jnp
from jax import lax
from jax.experimental import pallas as pl
from jax.experimental.pallas import tpu as pltpu

N_Z = 4
N_EXP_LOCAL = 2
D_MODEL = 1024
D_FF = 2048
T_PER = 1024
T_SUB = 512
A_COLS = 128


def kernel(x, assign, W1, W2):
    a_cols = jnp.broadcast_to(
        assign.astype(jnp.bfloat16)[:, None], (T_PER, A_COLS)
    )
    x_aug = jnp.concatenate([x.astype(jnp.bfloat16), a_cols], axis=1)
    w1b = W1.astype(jnp.bfloat16)
    w2b = W2.astype(jnp.bfloat16)

    def body(
        xaug_ref,
        w1_ref,
        w2_ref,
        out_ref,
        gather_ref,
        partials_ref,
        recv_ref,
        ag_send,
        ag_recv,
        rs_send,
        rs_recv,
        copy_sem,
    ):
        my_x = lax.axis_index("x")
        my_y = lax.axis_index("y")
        my_z = lax.axis_index("z")

        barrier_sem = pltpu.get_barrier_semaphore()
        for j in range(N_Z - 1):
            q = (my_z + 1 + j) % N_Z
            pl.semaphore_signal(
                barrier_sem,
                inc=1,
                device_id=(my_x, my_y, q),
                device_id_type=pl.DeviceIdType.MESH,
            )
        pl.semaphore_wait(barrier_sem, N_Z - 1)

        for j in range(N_Z - 1):
            q = (my_z + 1 + j) % N_Z
            r = pltpu.make_async_remote_copy(
                src_ref=xaug_ref,
                dst_ref=gather_ref.at[my_z],
                send_sem=ag_send.at[q],
                recv_sem=ag_recv.at[my_z],
                device_id=(my_x, my_y, q),
                device_id_type=pl.DeviceIdType.MESH,
            )
            r.start()

        cp = pltpu.make_async_copy(xaug_ref, gather_ref.at[my_z], copy_sem)
        cp.start()
        cp.wait()

        for j in range(N_Z - 1):
            p = (my_z + 1 + j) % N_Z
            w = pltpu.make_async_remote_copy(
                src_ref=xaug_ref,
                dst_ref=gather_ref.at[p],
                send_sem=ag_send.at[p],
                recv_sem=ag_recv.at[p],
                device_id=(my_x, my_y, p),
                device_id_type=pl.DeviceIdType.MESH,
            )
            w.wait_recv()

        for j in range(N_Z):
            c = (my_z + 1 + j) % N_Z
            xa = gather_ref[c]
            for tc in range(T_PER // T_SUB):
                rows = slice(tc * T_SUB, (tc + 1) * T_SUB)
                feat = xa[rows, :D_MODEL]
                acol = xa[rows, D_MODEL : D_MODEL + 1]
                acc = jnp.zeros((T_SUB, D_MODEL), dtype=jnp.float32)
                for le in range(N_EXP_LOCAL):
                    e = (N_EXP_LOCAL * my_z + le).astype(jnp.bfloat16)
                    mask = (acol == e).astype(jnp.bfloat16)
                    h = jnp.maximum(
                        jnp.dot(
                            feat * mask,
                            w1_ref[le],
                            preferred_element_type=jnp.float32,
                        ),
                        0.0,
                    ).astype(jnp.bfloat16)
                    acc = acc + jnp.dot(
                        h, w2_ref[le], preferred_element_type=jnp.float32
                    )
                if j < N_Z - 1:
                    partials_ref[j, rows, :] = acc.astype(jnp.bfloat16)
                else:
                    out_ref[rows, :] = acc
            if j < N_Z - 1:
                jr = (N_Z - 2) - j
                r = pltpu.make_async_remote_copy(
                    src_ref=partials_ref.at[j],
                    dst_ref=recv_ref.at[jr],
                    send_sem=rs_send.at[j],
                    recv_sem=rs_recv.at[jr],
                    device_id=(my_x, my_y, c),
                    device_id_type=pl.DeviceIdType.MESH,
                )
                r.start()

        total = out_ref[...]
        for j in range(N_Z - 1):
            p = (my_z + 1 + j) % N_Z
            w = pltpu.make_async_remote_copy(
                src_ref=partials_ref.at[j],
                dst_ref=recv_ref.at[j],
                send_sem=rs_send.at[j],
                recv_sem=rs_recv.at[j],
                device_id=(my_x, my_y, p),
                device_id_type=pl.DeviceIdType.MESH,
            )
            w.wait_recv()
            total = total + recv_ref[j].astype(jnp.float32)
        out_ref[...] = total

        for j in range(N_Z - 1):
            q = (my_z + 1 + j) % N_Z
            ws = pltpu.make_async_remote_copy(
                src_ref=xaug_ref,
                dst_ref=gather_ref.at[my_z],
                send_sem=ag_send.at[q],
                recv_sem=ag_recv.at[my_z],
                device_id=(my_x, my_y, q),
                device_id_type=pl.DeviceIdType.MESH,
            )
            ws.wait_send()
            wr = pltpu.make_async_remote_copy(
                src_ref=partials_ref.at[j],
                dst_ref=recv_ref.at[j],
                send_sem=rs_send.at[j],
                recv_sem=rs_recv.at[j],
                device_id=(my_x, my_y, q),
                device_id_type=pl.DeviceIdType.MESH,
            )
            wr.wait_send()

    out_shape = jax.ShapeDtypeStruct((T_PER, D_MODEL), jnp.float32)
    return pl.pallas_call(
        body,
        out_shape=out_shape,
        in_specs=[
            pl.BlockSpec(memory_space=pltpu.VMEM),
            pl.BlockSpec(memory_space=pltpu.VMEM),
            pl.BlockSpec(memory_space=pltpu.VMEM),
        ],
        out_specs=pl.BlockSpec(memory_space=pltpu.VMEM),
        scratch_shapes=[
            pltpu.VMEM((N_Z, T_PER, D_MODEL + A_COLS), jnp.bfloat16),
            pltpu.VMEM((N_Z - 1, T_PER, D_MODEL), jnp.bfloat16),
            pltpu.VMEM((N_Z - 1, T_PER, D_MODEL), jnp.bfloat16),
            pltpu.SemaphoreType.DMA((N_Z,)),
            pltpu.SemaphoreType.DMA((N_Z,)),
            pltpu.SemaphoreType.DMA((N_Z - 1,)),
            pltpu.SemaphoreType.DMA((N_Z - 1,)),
            pltpu.SemaphoreType.DMA,
        ],
        compiler_params=pltpu.CompilerParams(
            collective_id=0,
            vmem_limit_bytes=44 * 1024 * 1024,
        ),
    )(x_aug, w1b, w2b)
